# baseline (speedup 1.0000x reference)
import functools
import math

import jax
import jax.numpy as jnp
from jax import lax
from jax.experimental import pallas as pl

ALPHA = 0.5
F32 = jnp.float32
BF16 = jnp.bfloat16


def _dn(cl, cr):
    return (((cl,), (cr,)), ((), ()))


def _p2_body(x0_ref, inc_ref, wi_ref, b_ref,
             x_ref, nd_ref, deg_ref, x1t_ref, incb_ref, *, E, Q):
    i = pl.program_id(0)
    x_blk = jnp.maximum(
        lax.dot_general(x0_ref[...], wi_ref[...], _dn(1, 1),
                        preferred_element_type=F32) + b_ref[...], 0.0)
    x_ref[...] = x_blk
    inc_blk = inc_ref[...]
    nd_blk = jnp.sum(inc_blk, axis=1, keepdims=True)
    nd_ref[...] = nd_blk

    @pl.when(i == 0)
    def _():
        deg_ref[...] = jnp.zeros_like(deg_ref)
        x1t_ref[...] = jnp.zeros_like(x1t_ref)

    lhs = jnp.concatenate([jnp.ones_like(nd_blk), nd_blk], axis=1)
    deg_ref[...] += lax.dot_general(lhs, inc_blk, _dn(0, 0),
                                    preferred_element_type=F32)

    pad = 4 * Q - E
    if pad:
        incp = jnp.concatenate(
            [inc_blk, jnp.zeros((inc_blk.shape[0], pad), F32)], axis=1)
    else:
        incp = inc_blk
    xb = x_blk.astype(BF16)
    incb = incp.astype(BF16)
    incb_ref[...] = incb
    x1t_ref[...] += lax.dot_general(xb, incb, _dn(0, 0),
                                    preferred_element_type=F32)


def _p34_body(incb_ref, z1_ref, xskip_ref, rsnd_ref, w_ref,
              x2_ref, x1t_ref, *, beta, Q):
    i = pl.program_id(0)
    incb = incb_ref[...]
    m = lax.dot_general(incb, z1_ref[...], _dn(1, 0),
                        preferred_element_type=F32)
    m = m * rsnd_ref[...]
    xc = (1.0 - ALPHA) * m + ALPHA * xskip_ref[...]
    out = (1.0 - beta) * xc + beta * lax.dot_general(
        xc, w_ref[...], _dn(1, 1), preferred_element_type=F32)
    x2 = jnp.maximum(out, 0.0)
    x2_ref[...] = x2

    @pl.when(i == 0)
    def _():
        x1t_ref[...] = jnp.zeros_like(x1t_ref)

    x2b = x2.astype(BF16)
    x1t_ref[...] += lax.dot_general(x2b, incb, _dn(0, 0),
                                    preferred_element_type=F32)


def _p5_body(incb_ref, z_ref, xskip_ref, rsnd_ref, w_ref, out_ref, *, beta, Q):
    m = lax.dot_general(incb_ref[...], z_ref[...], _dn(1, 0),
                        preferred_element_type=F32)
    m = m * rsnd_ref[...]
    xc = (1.0 - ALPHA) * m + ALPHA * xskip_ref[...]
    out = (1.0 - beta) * xc + beta * lax.dot_general(
        xc, w_ref[...], _dn(1, 1), preferred_element_type=F32)
    out_ref[...] = jnp.maximum(out, 0.0)


def _pick_bn(n, cap):
    for bn in (400, 80, 64, 16):
        if bn <= cap and n % bn == 0:
            return bn
    return n


def kernel(x_0, incidence_1, W_init, b_init, W0, W1):
    N, D = x_0.shape
    E = incidence_1.shape[1]
    H = W_init.shape[0]
    BN2 = _pick_bn(N, 80)
    BN = _pick_bn(N, 400)
    Q = -(-E // (4 * 128)) * 128
    EP = 4 * Q
    b2 = b_init.reshape(1, H)

    x, nd, deg, x1t, packed = pl.pallas_call(
        functools.partial(_p2_body, E=E, Q=Q),
        grid=(N // BN2,),
        in_specs=[
            pl.BlockSpec((BN2, D), lambda i: (i, 0)),
            pl.BlockSpec((BN2, E), lambda i: (i, 0)),
            pl.BlockSpec((H, D), lambda i: (0, 0)),
            pl.BlockSpec((1, H), lambda i: (0, 0)),
        ],
        out_specs=[
            pl.BlockSpec((BN2, H), lambda i: (i, 0)),
            pl.BlockSpec((BN2, 1), lambda i: (i, 0)),
            pl.BlockSpec((2, E), lambda i: (0, 0)),
            pl.BlockSpec((H, EP), lambda i: (0, 0)),
            pl.BlockSpec((BN2, EP), lambda i: (i, 0)),
        ],
        out_shape=[
            jax.ShapeDtypeStruct((N, H), F32),
            jax.ShapeDtypeStruct((N, 1), F32),
            jax.ShapeDtypeStruct((2, E), F32),
            jax.ShapeDtypeStruct((H, EP), F32),
            jax.ShapeDtypeStruct((N, EP), BF16),
        ],
    )(x_0, incidence_1, W_init, b2)

    esz_row = deg[0:1]
    rsqe_row = lax.rsqrt(deg[1:2] / esz_row)
    scale_row = jnp.zeros((1, EP), F32).at[:, :E].set(rsqe_row / esz_row)
    z1 = (x1t * scale_row).T.astype(BF16)
    rsnd = lax.rsqrt(nd)

    beta1 = math.log(ALPHA / 1.0 + 1.0)
    beta2 = math.log(ALPHA / 2.0 + 1.0)

    x2, x1t2 = pl.pallas_call(
        functools.partial(_p34_body, beta=beta1, Q=Q),
        grid=(N // BN,),
        in_specs=[
            pl.BlockSpec((BN, EP), lambda i: (i, 0)),
            pl.BlockSpec((EP, H), lambda i: (0, 0)),
            pl.BlockSpec((BN, H), lambda i: (i, 0)),
            pl.BlockSpec((BN, 1), lambda i: (i, 0)),
            pl.BlockSpec((H, H), lambda i: (0, 0)),
        ],
        out_specs=[
            pl.BlockSpec((BN, H), lambda i: (i, 0)),
            pl.BlockSpec((H, EP), lambda i: (0, 0)),
        ],
        out_shape=[
            jax.ShapeDtypeStruct((N, H), F32),
            jax.ShapeDtypeStruct((H, EP), F32),
        ],
    )(packed, z1, x, rsnd, W0)

    x1_out = (x1t2[:, :E] / esz_row).T
    z2 = (x1t2 * scale_row).T.astype(BF16)

    x_out = pl.pallas_call(
        functools.partial(_p5_body, beta=beta2, Q=Q),
        grid=(N // BN,),
        in_specs=[
            pl.BlockSpec((BN, EP), lambda i: (i, 0)),
            pl.BlockSpec((EP, H), lambda i: (0, 0)),
            pl.BlockSpec((BN, H), lambda i: (i, 0)),
            pl.BlockSpec((BN, 1), lambda i: (i, 0)),
            pl.BlockSpec((H, H), lambda i: (0, 0)),
        ],
        out_specs=pl.BlockSpec((BN, H), lambda i: (i, 0)),
        out_shape=jax.ShapeDtypeStruct((N, H), F32),
    )(packed, z2, x, rsnd, W1)

    return x_out, x1_out


# speedup vs baseline: 1.2652x; 1.2652x over previous
import functools
import math

import jax
import jax.numpy as jnp
from jax import lax
from jax.experimental import pallas as pl

ALPHA = 0.5
F32 = jnp.float32
BF16 = jnp.bfloat16


def _dn(cl, cr):
    return (((cl,), (cr,)), ((), ()))


def _p2_body(x0_ref, inc_ref, wi_ref, b_ref,
             x_ref, nd_ref, deg_ref, x1t_ref, pk_ref, *, E, Q):
    i = pl.program_id(0)
    x_blk = jnp.maximum(
        lax.dot_general(x0_ref[...], wi_ref[...], _dn(1, 1),
                        preferred_element_type=F32) + b_ref[...], 0.0)
    x_ref[...] = x_blk
    inc_blk = inc_ref[...]
    nd_blk = jnp.sum(inc_blk, axis=1, keepdims=True)
    nd_ref[...] = nd_blk

    @pl.when(i == 0)
    def _():
        deg_ref[...] = jnp.zeros_like(deg_ref)
        x1t_ref[...] = jnp.zeros_like(x1t_ref)

    lhs = jnp.concatenate([jnp.ones_like(nd_blk), nd_blk], axis=1)
    deg_ref[...] += lax.dot_general(lhs, inc_blk, _dn(0, 0),
                                    preferred_element_type=F32)

    pad = 4 * Q - E
    if pad:
        incp = jnp.concatenate(
            [inc_blk, jnp.zeros((inc_blk.shape[0], pad), F32)], axis=1)
    else:
        incp = inc_blk
    pk = jnp.zeros(incp.shape[:1] + (Q,), jnp.int32)
    xb = x_blk.astype(BF16)
    for k in range(4):
        qk = incp[:, k * Q:(k + 1) * Q]
        pk = pk | (qk.astype(jnp.int32) << (8 * k))
        x1t_ref[:, k * Q:(k + 1) * Q] += lax.dot_general(
            xb, qk.astype(BF16), _dn(0, 0), preferred_element_type=F32)
    pk_ref[...] = pk


def _p34_body(pk_ref, z1_ref, xskip_ref, rsnd_ref, w_ref,
              x2_ref, x1t_ref, *, beta, Q):
    i = pl.program_id(0)
    pk = pk_ref[...]
    m = jnp.zeros(x2_ref.shape, F32)
    for k in range(4):
        qk = ((pk >> (8 * k)) & 0xFF).astype(F32).astype(BF16)
        m += lax.dot_general(qk, z1_ref[k * Q:(k + 1) * Q, :], _dn(1, 0),
                             preferred_element_type=F32)
    m = m * rsnd_ref[...]
    xc = (1.0 - ALPHA) * m + ALPHA * xskip_ref[...]
    out = (1.0 - beta) * xc + beta * lax.dot_general(
        xc, w_ref[...], _dn(1, 1), preferred_element_type=F32)
    x2 = jnp.maximum(out, 0.0)
    x2_ref[...] = x2

    @pl.when(i == 0)
    def _():
        x1t_ref[...] = jnp.zeros_like(x1t_ref)

    x2b = x2.astype(BF16)
    for k in range(4):
        qk = ((pk >> (8 * k)) & 0xFF).astype(F32).astype(BF16)
        x1t_ref[:, k * Q:(k + 1) * Q] += lax.dot_general(
            x2b, qk, _dn(0, 0), preferred_element_type=F32)


def _p5_body(pk_ref, z_ref, xskip_ref, rsnd_ref, w_ref, out_ref, *, beta, Q):
    pk = pk_ref[...]
    m = jnp.zeros(out_ref.shape, F32)
    for k in range(4):
        qk = ((pk >> (8 * k)) & 0xFF).astype(F32).astype(BF16)
        m += lax.dot_general(qk, z_ref[k * Q:(k + 1) * Q, :], _dn(1, 0),
                             preferred_element_type=F32)
    m = m * rsnd_ref[...]
    xc = (1.0 - ALPHA) * m + ALPHA * xskip_ref[...]
    out = (1.0 - beta) * xc + beta * lax.dot_general(
        xc, w_ref[...], _dn(1, 1), preferred_element_type=F32)
    out_ref[...] = jnp.maximum(out, 0.0)


def _pick_bn(n, cap):
    for bn in (1000, 400, 200, 128, 80, 64, 40, 16, 8):
        if bn <= cap and n % bn == 0:
            return bn
    return n


def kernel(x_0, incidence_1, W_init, b_init, W0, W1):
    N, D = x_0.shape
    E = incidence_1.shape[1]
    H = W_init.shape[0]
    BN2 = _pick_bn(N, 200)
    BN = _pick_bn(N, 1000)
    Q = -(-E // (4 * 128)) * 128
    EP = 4 * Q
    b2 = b_init.reshape(1, H)

    x, nd, deg, x1t, packed = pl.pallas_call(
        functools.partial(_p2_body, E=E, Q=Q),
        grid=(N // BN2,),
        in_specs=[
            pl.BlockSpec((BN2, D), lambda i: (i, 0)),
            pl.BlockSpec((BN2, E), lambda i: (i, 0)),
            pl.BlockSpec((H, D), lambda i: (0, 0)),
            pl.BlockSpec((1, H), lambda i: (0, 0)),
        ],
        out_specs=[
            pl.BlockSpec((BN2, H), lambda i: (i, 0)),
            pl.BlockSpec((BN2, 1), lambda i: (i, 0)),
            pl.BlockSpec((2, E), lambda i: (0, 0)),
            pl.BlockSpec((H, EP), lambda i: (0, 0)),
            pl.BlockSpec((BN2, Q), lambda i: (i, 0)),
        ],
        out_shape=[
            jax.ShapeDtypeStruct((N, H), F32),
            jax.ShapeDtypeStruct((N, 1), F32),
            jax.ShapeDtypeStruct((2, E), F32),
            jax.ShapeDtypeStruct((H, EP), F32),
            jax.ShapeDtypeStruct((N, Q), jnp.int32),
        ],
    )(x_0, incidence_1, W_init, b2)

    esz_row = deg[0:1]
    rsqe_row = lax.rsqrt(deg[1:2] / esz_row)
    scale_row = jnp.zeros((1, EP), F32).at[:, :E].set(rsqe_row / esz_row)
    z1 = (x1t * scale_row).T.astype(BF16)
    rsnd = lax.rsqrt(nd)

    beta1 = math.log(ALPHA / 1.0 + 1.0)
    beta2 = math.log(ALPHA / 2.0 + 1.0)

    x2, x1t2 = pl.pallas_call(
        functools.partial(_p34_body, beta=beta1, Q=Q),
        grid=(N // BN,),
        in_specs=[
            pl.BlockSpec((BN, Q), lambda i: (i, 0)),
            pl.BlockSpec((EP, H), lambda i: (0, 0)),
            pl.BlockSpec((BN, H), lambda i: (i, 0)),
            pl.BlockSpec((BN, 1), lambda i: (i, 0)),
            pl.BlockSpec((H, H), lambda i: (0, 0)),
        ],
        out_specs=[
            pl.BlockSpec((BN, H), lambda i: (i, 0)),
            pl.BlockSpec((H, EP), lambda i: (0, 0)),
        ],
        out_shape=[
            jax.ShapeDtypeStruct((N, H), F32),
            jax.ShapeDtypeStruct((H, EP), F32),
        ],
    )(packed, z1, x, rsnd, W0)

    x1_out = (x1t2[:, :E] / esz_row).T
    z2 = (x1t2 * scale_row).T.astype(BF16)

    x_out = pl.pallas_call(
        functools.partial(_p5_body, beta=beta2, Q=Q),
        grid=(N // BN,),
        in_specs=[
            pl.BlockSpec((BN, Q), lambda i: (i, 0)),
            pl.BlockSpec((EP, H), lambda i: (0, 0)),
            pl.BlockSpec((BN, H), lambda i: (i, 0)),
            pl.BlockSpec((BN, 1), lambda i: (i, 0)),
            pl.BlockSpec((H, H), lambda i: (0, 0)),
        ],
        out_specs=pl.BlockSpec((BN, H), lambda i: (i, 0)),
        out_shape=jax.ShapeDtypeStruct((N, H), F32),
    )(packed, z2, x, rsnd, W1)

    return x_out, x1_out


# speedup vs baseline: 1.3228x; 1.0455x over previous
import functools
import math

import jax
import jax.numpy as jnp
from jax import lax
from jax.experimental import pallas as pl

ALPHA = 0.5
F32 = jnp.float32
BF16 = jnp.bfloat16


def _dn(cl, cr):
    return (((cl,), (cr,)), ((), ()))


def _p2_body(x0_ref, inc_ref, wi_ref, b_ref,
             x_ref, nd_ref, deg_ref, x1t_ref, pk_ref, *, E, Q):
    i = pl.program_id(0)
    x_blk = jnp.maximum(
        lax.dot_general(x0_ref[...], wi_ref[...], _dn(1, 1),
                        preferred_element_type=F32) + b_ref[...], 0.0)
    x_ref[...] = x_blk
    inc_blk = inc_ref[...]
    nd_blk = jnp.sum(inc_blk, axis=1, keepdims=True)
    nd_ref[...] = nd_blk

    @pl.when(i == 0)
    def _():
        deg_ref[...] = jnp.zeros_like(deg_ref)
        x1t_ref[...] = jnp.zeros_like(x1t_ref)

    lhs = jnp.concatenate([jnp.ones_like(nd_blk), nd_blk], axis=1)
    deg_ref[...] += lax.dot_general(lhs, inc_blk, _dn(0, 0),
                                    preferred_element_type=F32)

    pad = 4 * Q - E
    if pad:
        incp = jnp.concatenate(
            [inc_blk, jnp.zeros((inc_blk.shape[0], pad), F32)], axis=1)
    else:
        incp = inc_blk
    pk = jnp.zeros(incp.shape[:1] + (Q,), jnp.int32)
    xb = x_blk.astype(BF16)
    for k in range(4):
        qk = incp[:, k * Q:(k + 1) * Q]
        pk = pk | (qk.astype(jnp.int32) << (8 * k))
        x1t_ref[:, k * Q:(k + 1) * Q] += lax.dot_general(
            xb, qk.astype(BF16), _dn(0, 0), preferred_element_type=F32)
    pk_ref[...] = pk


def _p4_body(pkc_ref, x2t_ref, x1q_ref):
    pkc = pkc_ref[...]
    x2t = x2t_ref[...]
    for k in range(4):
        qk = ((pkc >> (8 * k)) & 0xFF).astype(F32).astype(BF16)
        x1q_ref[k, :, :] = lax.dot_general(x2t, qk, _dn(1, 0),
                                           preferred_element_type=F32)


def _p5_body(pk_ref, z_ref, xskip_ref, rsnd_ref, w_ref, out_ref, *, beta, Q):
    pk = pk_ref[...]
    m = jnp.zeros(out_ref.shape, F32)
    for k in range(4):
        qk = ((pk >> (8 * k)) & 0xFF).astype(F32).astype(BF16)
        m += lax.dot_general(qk, z_ref[k * Q:(k + 1) * Q, :], _dn(1, 0),
                             preferred_element_type=F32)
    m = m * rsnd_ref[...]
    xc = (1.0 - ALPHA) * m + ALPHA * xskip_ref[...]
    out = (1.0 - beta) * xc + beta * lax.dot_general(
        xc, w_ref[...], _dn(1, 1), preferred_element_type=F32)
    out_ref[...] = jnp.maximum(out, 0.0)


def _pick_bn(n, cap):
    for bn in (1000, 400, 200, 128, 80, 64, 40, 16, 8):
        if bn <= cap and n % bn == 0:
            return bn
    return n


def kernel(x_0, incidence_1, W_init, b_init, W0, W1):
    N, D = x_0.shape
    E = incidence_1.shape[1]
    H = W_init.shape[0]
    BN2 = _pick_bn(N, 200)
    BN = _pick_bn(N, 1000)
    Q = -(-E // (4 * 128)) * 128
    EP = 4 * Q
    b2 = b_init.reshape(1, H)

    x, nd, deg, x1t, packed = pl.pallas_call(
        functools.partial(_p2_body, E=E, Q=Q),
        grid=(N // BN2,),
        in_specs=[
            pl.BlockSpec((BN2, D), lambda i: (i, 0)),
            pl.BlockSpec((BN2, E), lambda i: (i, 0)),
            pl.BlockSpec((H, D), lambda i: (0, 0)),
            pl.BlockSpec((1, H), lambda i: (0, 0)),
        ],
        out_specs=[
            pl.BlockSpec((BN2, H), lambda i: (i, 0)),
            pl.BlockSpec((BN2, 1), lambda i: (i, 0)),
            pl.BlockSpec((2, E), lambda i: (0, 0)),
            pl.BlockSpec((H, EP), lambda i: (0, 0)),
            pl.BlockSpec((BN2, Q), lambda i: (i, 0)),
        ],
        out_shape=[
            jax.ShapeDtypeStruct((N, H), F32),
            jax.ShapeDtypeStruct((N, 1), F32),
            jax.ShapeDtypeStruct((2, E), F32),
            jax.ShapeDtypeStruct((H, EP), F32),
            jax.ShapeDtypeStruct((N, Q), jnp.int32),
        ],
    )(x_0, incidence_1, W_init, b2)

    esz_row = deg[0:1]
    rsqe_row = lax.rsqrt(deg[1:2] / esz_row)
    scale_row = jnp.zeros((1, EP), F32).at[:, :E].set(rsqe_row / esz_row)
    z1 = (x1t * scale_row).T.astype(BF16)
    rsnd = lax.rsqrt(nd)

    beta1 = math.log(ALPHA / 1.0 + 1.0)
    beta2 = math.log(ALPHA / 2.0 + 1.0)

    def m_pass(z, W, beta):
        return pl.pallas_call(
            functools.partial(_p5_body, beta=beta, Q=Q),
            grid=(N // BN,),
            in_specs=[
                pl.BlockSpec((BN, Q), lambda i: (i, 0)),
                pl.BlockSpec((EP, H), lambda i: (0, 0)),
                pl.BlockSpec((BN, H), lambda i: (i, 0)),
                pl.BlockSpec((BN, 1), lambda i: (i, 0)),
                pl.BlockSpec((H, H), lambda i: (0, 0)),
            ],
            out_specs=pl.BlockSpec((BN, H), lambda i: (i, 0)),
            out_shape=jax.ShapeDtypeStruct((N, H), F32),
        )(packed, z, x, rsnd, W)

    x2 = m_pass(z1, W0, beta1)
    x2t = x2.T.astype(BF16)

    QB = 256 if Q % 256 == 0 else Q
    x1q = pl.pallas_call(
        _p4_body,
        grid=(Q // QB,),
        in_specs=[
            pl.BlockSpec((N, QB), lambda i: (0, i)),
            pl.BlockSpec((H, N), lambda i: (0, 0)),
        ],
        out_specs=pl.BlockSpec((4, H, QB), lambda i: (0, 0, i)),
        out_shape=jax.ShapeDtypeStruct((4, H, Q), F32),
    )(packed, x2t)
    x1t2 = jnp.swapaxes(x1q, 0, 1).reshape(H, EP)

    x1_out = (x1t2[:, :E] / esz_row).T
    z2 = (x1t2 * scale_row).T.astype(BF16)

    x_out = m_pass(z2, W1, beta2)
    return x_out, x1_out


# speedup vs baseline: 1.3935x; 1.0534x over previous
import functools
import math

import jax
import jax.numpy as jnp
from jax import lax
from jax.experimental import pallas as pl

ALPHA = 0.5
F32 = jnp.float32
BF16 = jnp.bfloat16


def _dn(cl, cr):
    return (((cl,), (cr,)), ((), ()))


def _p2_body(x0_ref, inc_ref, wi_ref, b_ref,
             x_ref, nd_ref, deg_ref, x1t_ref, pk_ref, *, E, EP):
    i = pl.program_id(0)
    x_blk = jnp.maximum(
        lax.dot_general(x0_ref[...], wi_ref[...], _dn(1, 1),
                        preferred_element_type=F32) + b_ref[...], 0.0)
    x_ref[...] = x_blk
    inc_blk = inc_ref[...]
    nd_blk = jnp.sum(inc_blk, axis=1, keepdims=True)
    nd_ref[...] = nd_blk

    @pl.when(i == 0)
    def _():
        deg_ref[...] = jnp.zeros_like(deg_ref)
        x1t_ref[...] = jnp.zeros_like(x1t_ref)

    lhs = jnp.concatenate([jnp.ones_like(nd_blk), nd_blk], axis=1)
    deg_ref[...] += lax.dot_general(lhs, inc_blk, _dn(0, 0),
                                    preferred_element_type=F32)

    pad = EP - E
    if pad:
        incp = jnp.concatenate(
            [inc_blk, jnp.zeros((inc_blk.shape[0], pad), F32)], axis=1)
    else:
        incp = inc_blk
    bn = incp.shape[0]
    pk_ref[...] = incp.astype(jnp.int8).reshape(bn // 8, 8, EP)
    x1t_ref[...] += lax.dot_general(
        x_blk.astype(BF16), incp.astype(BF16), _dn(0, 0),
        preferred_element_type=F32)


def _m_body(pk_ref, z_ref, xskip_ref, rsnd_ref, w_ref, out_ref, *, beta):
    pk = pk_ref[...]
    bn = pk.shape[0] * 8
    q = pk.reshape(bn, pk.shape[2]).astype(BF16)
    m = lax.dot_general(q, z_ref[...], _dn(1, 0),
                        preferred_element_type=F32)
    m = m * rsnd_ref[...]
    xc = (1.0 - ALPHA) * m + ALPHA * xskip_ref[...]
    out = (1.0 - beta) * xc + beta * lax.dot_general(
        xc, w_ref[...], _dn(1, 1), preferred_element_type=F32)
    out_ref[...] = jnp.maximum(out, 0.0)


def _p4_body(pkc_ref, x2t_ref, x1t_ref):
    pkc = pkc_ref[...]
    n = pkc.shape[0] * 8
    q = pkc.reshape(n, pkc.shape[2]).astype(BF16)
    x1t_ref[...] = lax.dot_general(x2t_ref[...], q, _dn(1, 0),
                                   preferred_element_type=F32)


def _pick_bn(n, cap):
    for bn in (1000, 400, 200, 128, 80, 64, 40, 16, 8):
        if bn <= cap and n % bn == 0:
            return bn
    return n


def kernel(x_0, incidence_1, W_init, b_init, W0, W1):
    N, D = x_0.shape
    E = incidence_1.shape[1]
    H = W_init.shape[0]
    BN2 = _pick_bn(N, 200)
    BN = _pick_bn(N, 1000)
    EP = -(-E // 512) * 512
    b2 = b_init.reshape(1, H)

    x, nd, deg, x1t, packed = pl.pallas_call(
        functools.partial(_p2_body, E=E, EP=EP),
        grid=(N // BN2,),
        in_specs=[
            pl.BlockSpec((BN2, D), lambda i: (i, 0)),
            pl.BlockSpec((BN2, E), lambda i: (i, 0)),
            pl.BlockSpec((H, D), lambda i: (0, 0)),
            pl.BlockSpec((1, H), lambda i: (0, 0)),
        ],
        out_specs=[
            pl.BlockSpec((BN2, H), lambda i: (i, 0)),
            pl.BlockSpec((BN2, 1), lambda i: (i, 0)),
            pl.BlockSpec((2, E), lambda i: (0, 0)),
            pl.BlockSpec((H, EP), lambda i: (0, 0)),
            pl.BlockSpec((BN2 // 8, 8, EP), lambda i: (i, 0, 0)),
        ],
        out_shape=[
            jax.ShapeDtypeStruct((N, H), F32),
            jax.ShapeDtypeStruct((N, 1), F32),
            jax.ShapeDtypeStruct((2, E), F32),
            jax.ShapeDtypeStruct((H, EP), F32),
            jax.ShapeDtypeStruct((N // 8, 8, EP), jnp.int8),
        ],
    )(x_0, incidence_1, W_init, b2)

    esz_row = deg[0:1]
    rsqe_row = lax.rsqrt(deg[1:2] / esz_row)
    scale_row = jnp.zeros((1, EP), F32).at[:, :E].set(rsqe_row / esz_row)
    z1 = (x1t * scale_row).T.astype(BF16)
    rsnd = lax.rsqrt(nd)

    beta1 = math.log(ALPHA / 1.0 + 1.0)
    beta2 = math.log(ALPHA / 2.0 + 1.0)

    def m_pass(z, W, beta):
        return pl.pallas_call(
            functools.partial(_m_body, beta=beta),
            grid=(N // BN,),
            in_specs=[
                pl.BlockSpec((BN // 8, 8, EP), lambda i: (i, 0, 0)),
                pl.BlockSpec((EP, H), lambda i: (0, 0)),
                pl.BlockSpec((BN, H), lambda i: (i, 0)),
                pl.BlockSpec((BN, 1), lambda i: (i, 0)),
                pl.BlockSpec((H, H), lambda i: (0, 0)),
            ],
            out_specs=pl.BlockSpec((BN, H), lambda i: (i, 0)),
            out_shape=jax.ShapeDtypeStruct((N, H), F32),
        )(packed, z, x, rsnd, W)

    x2 = m_pass(z1, W0, beta1)
    x2t = x2.T.astype(BF16)

    QB = 512 if EP % 512 == 0 else EP
    x1t2 = pl.pallas_call(
        _p4_body,
        grid=(EP // QB,),
        in_specs=[
            pl.BlockSpec((N // 8, 8, QB), lambda i: (0, 0, i)),
            pl.BlockSpec((H, N), lambda i: (0, 0)),
        ],
        out_specs=pl.BlockSpec((H, QB), lambda i: (0, i)),
        out_shape=jax.ShapeDtypeStruct((H, EP), F32),
    )(packed, x2t)

    x1_out = (x1t2[:, :E] / esz_row).T
    z2 = (x1t2 * scale_row).T.astype(BF16)

    x_out = m_pass(z2, W1, beta2)
    return x_out, x1_out
